# baseline (speedup 1.0000x reference)
import jax
import jax.numpy as jnp
from jax import lax
from jax.experimental import pallas as pl
from jax.experimental.pallas import tpu as pltpu
from jax.experimental.pallas import tpu_sc as plsc

_VOCAB = 1000000
_HIDDEN = 64
_BATCH = 16384
_HIST = 200
_B = _BATCH * _HIST

_NC = 2
_NS = 16
_NW = _NC * _NS
_BPW = _B // _NW
_C = 1600
_NCHUNK = _BPW // _C


def _body(x_hbm, w_hbm, out_hbm, idx_v, rows_v, sem):
    wid = lax.axis_index("s") * _NC + lax.axis_index("c")
    base = wid * _BPW

    def step(i, carry):
        off = base + i * _C
        pltpu.sync_copy(x_hbm.at[pl.ds(off, _C)], idx_v)
        pltpu.async_copy(w_hbm.at[idx_v], rows_v, sem).wait()
        pltpu.sync_copy(rows_v, out_hbm.at[pl.ds(off, _C)])
        return carry

    lax.fori_loop(0, _NCHUNK, step, 0)


def kernel(x, weight):
    xf = x.reshape(-1).astype(jnp.int32)
    mesh = plsc.VectorSubcoreMesh(
        core_axis_name="c", subcore_axis_name="s",
        num_cores=_NC, num_subcores=_NS)
    out = pl.kernel(
        _body,
        out_type=jax.ShapeDtypeStruct((_B, _HIDDEN), jnp.float32),
        mesh=mesh,
        compiler_params=pltpu.CompilerParams(use_tc_tiling_on_sc=False),
        scratch_types=[
            pltpu.VMEM((_C,), jnp.int32),
            pltpu.VMEM((_C, _HIDDEN), jnp.float32),
            pltpu.SemaphoreType.DMA,
        ],
    )(xf, weight)
    return out.reshape(_BATCH, _HIST, _HIDDEN)


# speedup vs baseline: 1.0035x; 1.0035x over previous
import jax
import jax.numpy as jnp
from jax import lax
from jax.experimental import pallas as pl
from jax.experimental.pallas import tpu as pltpu
from jax.experimental.pallas import tpu_sc as plsc

_VOCAB = 1000000
_HIDDEN = 64
_BATCH = 16384
_HIST = 200
_B = _BATCH * _HIST

_NC = 2
_NS = 16
_NW = _NC * _NS
_BPW = _B // _NW
_C = 800
_NCHUNK = _BPW // _C


def _body(x_hbm, w_hbm, out_hbm,
          idx0, idx1, rows0, rows1, sg0, sg1, sw0, sw1):
    wid = lax.axis_index("s") * _NC + lax.axis_index("c")
    base = wid * _BPW

    def start_chunk(idx_v, rows_v, sg, c):
        pltpu.sync_copy(x_hbm.at[pl.ds(base + c * _C, _C)], idx_v)
        pltpu.async_copy(w_hbm.at[idx_v], rows_v, sg)

    def wait_gather(idx_v, rows_v, sg):
        pltpu.make_async_copy(w_hbm.at[idx_v], rows_v, sg).wait()

    def start_write(rows_v, sw, c):
        pltpu.async_copy(rows_v, out_hbm.at[pl.ds(base + c * _C, _C)], sw)

    def wait_write(rows_v, sw, c):
        pltpu.make_async_copy(rows_v, out_hbm.at[pl.ds(base + c * _C, _C)],
                              sw).wait()

    start_chunk(idx0, rows0, sg0, 0)
    start_chunk(idx1, rows1, sg1, 1)

    def step(j, carry):
        c0 = j * 2

        wait_gather(idx0, rows0, sg0)
        start_write(rows0, sw0, c0)

        @pl.when(c0 + 2 < _NCHUNK)
        def _():
            wait_write(rows0, sw0, c0)
            start_chunk(idx0, rows0, sg0, c0 + 2)

        wait_gather(idx1, rows1, sg1)
        start_write(rows1, sw1, c0 + 1)

        @pl.when(c0 + 3 < _NCHUNK)
        def _():
            wait_write(rows1, sw1, c0 + 1)
            start_chunk(idx1, rows1, sg1, c0 + 3)

        return carry

    lax.fori_loop(0, _NCHUNK // 2, step, 0)

    wait_write(rows0, sw0, _NCHUNK - 2)
    wait_write(rows1, sw1, _NCHUNK - 1)


def kernel(x, weight):
    xf = x.reshape(-1).astype(jnp.int32)
    mesh = plsc.VectorSubcoreMesh(
        core_axis_name="c", subcore_axis_name="s",
        num_cores=_NC, num_subcores=_NS)
    out = pl.kernel(
        _body,
        out_type=jax.ShapeDtypeStruct((_B, _HIDDEN), jnp.float32),
        mesh=mesh,
        compiler_params=pltpu.CompilerParams(use_tc_tiling_on_sc=False),
        scratch_types=[
            pltpu.VMEM((_C,), jnp.int32),
            pltpu.VMEM((_C,), jnp.int32),
            pltpu.VMEM((_C, _HIDDEN), jnp.float32),
            pltpu.VMEM((_C, _HIDDEN), jnp.float32),
            pltpu.SemaphoreType.DMA,
            pltpu.SemaphoreType.DMA,
            pltpu.SemaphoreType.DMA,
            pltpu.SemaphoreType.DMA,
        ],
    )(xf, weight)
    return out.reshape(_BATCH, _HIST, _HIDDEN)
